# baseline (speedup 1.0000x reference)
import functools
import math

import jax
import jax.numpy as jnp
from jax import lax
from jax.experimental import pallas as pl
from jax.experimental.pallas import tpu as pltpu
from jax.experimental.pallas import tpu_sc as plsc

D_MODEL = 64
SCALE = math.sqrt(D_MODEL)
LANES = 16
CHUNK = 640
ROW_UNROLL = 4


@functools.lru_cache(maxsize=None)
def _make_emb(B):
    info = plsc.get_sparse_core_info()
    nw = info.num_cores * info.num_subcores
    b_per_w = B // nw
    n_chunks = b_per_w // CHUNK
    assert n_chunks % 2 == 0
    mesh = plsc.VectorSubcoreMesh(core_axis_name="c", subcore_axis_name="s")

    @functools.partial(
        pl.kernel,
        mesh=mesh,
        out_type=jax.ShapeDtypeStruct((B, D_MODEL), jnp.float32),
        scratch_types=[
            pltpu.VMEM((n_chunks, CHUNK), jnp.int32),
            pltpu.VMEM((CHUNK, D_MODEL), jnp.float32),
            pltpu.VMEM((CHUNK, D_MODEL), jnp.float32),
            pltpu.SemaphoreType.DMA,
            pltpu.SemaphoreType.DMA,
        ],
        compiler_params=pltpu.CompilerParams(use_tc_tiling_on_sc=False),
    )
    def emb(x_hbm, table_hbm, out_hbm, idx_v, rows0, rows1, sem0, sem1):
        wid = lax.axis_index("s") * info.num_cores + lax.axis_index("c")
        wbase = wid * b_per_w
        bufs = (rows0, rows1)
        sems = (sem0, sem1)

        pltpu.sync_copy(x_hbm.at[wid], idx_v)

        pltpu.async_copy(table_hbm.at[idx_v.at[0]], rows0, sem0)
        pltpu.async_copy(table_hbm.at[idx_v.at[1]], rows1, sem1)

        def group_body(g, carry):
            for b in range(2):
                ci = g * 2 + b
                buf = bufs[b]
                pltpu.make_async_copy(table_hbm.at[idx_v.at[ci]], buf, sems[b]).wait()

                def scale_rows(rq, c2):
                    r0 = rq * ROW_UNROLL
                    for rr in range(ROW_UNROLL):
                        for c in range(D_MODEL // LANES):
                            sl = pl.ds(c * LANES, LANES)
                            buf[r0 + rr, sl] = buf[r0 + rr, sl] * SCALE
                    return c2

                lax.fori_loop(0, CHUNK // ROW_UNROLL, scale_rows, 0)

                pltpu.sync_copy(buf, out_hbm.at[pl.ds(wbase + ci * CHUNK, CHUNK)])

                @pl.when(ci + 2 < n_chunks)
                def _():
                    pltpu.async_copy(table_hbm.at[idx_v.at[ci + 2]], buf, sems[b])

            return carry

        lax.fori_loop(0, n_chunks // 2, group_body, 0)

    return emb


def kernel(x, table):
    B = x.shape[0] * x.shape[1]
    info = plsc.get_sparse_core_info()
    nw = info.num_cores * info.num_subcores
    b_per_w = B // nw
    x_r = x.reshape((nw, b_per_w // CHUNK, CHUNK)).astype(jnp.int32)
    out = _make_emb(B)(x_r, table)
    return out.reshape(x.shape + (D_MODEL,))


# speedup vs baseline: 1.0038x; 1.0038x over previous
import functools
import math

import jax
import jax.numpy as jnp
from jax import lax
from jax.experimental import pallas as pl
from jax.experimental.pallas import tpu as pltpu
from jax.experimental.pallas import tpu_sc as plsc

D_MODEL = 64
SCALE = math.sqrt(D_MODEL)
LANES = 16
NBUF = 4
ROW_UNROLL = 4


@functools.lru_cache(maxsize=None)
def _make_emb(R, T):
    info = plsc.get_sparse_core_info()
    nw = info.num_cores * info.num_subcores
    r_per_w = R // nw
    mesh = plsc.VectorSubcoreMesh(core_axis_name="c", subcore_axis_name="s")

    @functools.partial(
        pl.kernel,
        mesh=mesh,
        out_type=jax.ShapeDtypeStruct((R, T, D_MODEL), jnp.float32),
        scratch_types=[
            pltpu.VMEM((r_per_w, T), jnp.int32),
            *[pltpu.VMEM((T, D_MODEL), jnp.float32) for _ in range(NBUF)],
            *[pltpu.SemaphoreType.DMA for _ in range(NBUF)],
        ],
        compiler_params=pltpu.CompilerParams(use_tc_tiling_on_sc=False),
    )
    def emb(x_hbm, table_hbm, out_hbm, idx_v, *bufs_sems):
        bufs = bufs_sems[:NBUF]
        sems = bufs_sems[NBUF:]
        wid = lax.axis_index("s") * info.num_cores + lax.axis_index("c")
        r_base = wid * r_per_w

        pltpu.sync_copy(x_hbm.at[pl.ds(r_base, r_per_w)], idx_v)

        for b in range(NBUF):
            pltpu.async_copy(table_hbm.at[idx_v.at[b]], bufs[b], sems[b])

        def group_body(g, carry):
            for b in range(NBUF):
                j = g * NBUF + b
                buf = bufs[b]
                pltpu.make_async_copy(
                    table_hbm.at[idx_v.at[j]], buf, sems[b]
                ).wait()

                def scale_rows(rq, c2):
                    r0 = rq * ROW_UNROLL
                    for rr in range(ROW_UNROLL):
                        for c in range(D_MODEL // LANES):
                            sl = pl.ds(c * LANES, LANES)
                            buf[r0 + rr, sl] = buf[r0 + rr, sl] * SCALE
                    return c2

                lax.fori_loop(0, T // ROW_UNROLL, scale_rows, 0)

                pltpu.sync_copy(buf, out_hbm.at[r_base + j])

                @pl.when(j + NBUF < r_per_w)
                def _():
                    pltpu.async_copy(table_hbm.at[idx_v.at[j + NBUF]], buf, sems[b])

            return carry

        lax.fori_loop(0, r_per_w // NBUF, group_body, 0)

    return emb


def kernel(x, table):
    return _make_emb(x.shape[0], x.shape[1])(x.astype(jnp.int32), table)
